# baseline (speedup 1.0000x reference)
import jax
import jax.numpy as jnp
from jax import lax
from jax.experimental import pallas as pl
from jax.experimental.pallas import tpu as pltpu
from jax.experimental.pallas import tpu_sc as plsc

NC, NS, L = 2, 16, 16
NW = NC * NS
K = 3
WO = 180
WP = 192
NACC = 12
NCHUNK = WO // NACC
WROW = WO + NACC
NLAT_W = 5
BC_CH = 16


def _sc_sparse_stage(xpar, off, val, ptr2d, BC, H, HO):
    S16P = off.shape[0]
    BCJ = BC // BC_CH
    NJOBS = HO * BCJ
    NJ = -(-NJOBS // NW)
    NJ += NJ % 2

    mesh = plsc.VectorSubcoreMesh(core_axis_name="c", subcore_axis_name="s",
                                  num_cores=NC, num_subcores=NS)

    def body(xpar_hbm, off_hbm, val_hbm, ptr_hbm, y_hbm,
             off_v, val_v, ptr_v, xw, stage_v):
        wid = lax.axis_index("s") * NC + lax.axis_index("c")
        pltpu.sync_copy(off_hbm, off_v)
        pltpu.sync_copy(val_hbm, val_v)
        pltpu.sync_copy(ptr_hbm, ptr_v)

        def compute(j):
            i = j // BCJ
            cb = j % BCJ
            lat0 = jnp.clip(2 * i - 2, 0, H - NLAT_W)
            pltpu.sync_copy(xpar_hbm.at[cb, pl.ds(2 * lat0, 2 * NLAT_W)], xw)
            pr = ptr_v[i]

            def chunk(pc, carry2):
                po0 = pc * NACC
                for k in range(K):
                    t0 = pr[k]
                    nb = (pr[k + 1] - t0) >> 4

                    def blk(b, acc):
                        base = t0 + b * L
                        offv = off_v[pl.ds(base, L)]
                        valv = val_v[pl.ds(base, L)]
                        for li in range(L):
                            o = offv[li]
                            v = valv[li]
                            ab = o >> 9
                            mp = (o & 511) + po0
                            mp = jnp.where(mp >= WO, mp - WO, mp)
                            acc = tuple(
                                acc[wv] + v * xw[ab, mp + wv, :]
                                for wv in range(NACC))
                        return acc

                    acc0 = tuple(jnp.zeros((L,), jnp.float32)
                                 for _ in range(NACC))
                    acc = lax.fori_loop(0, nb, blk, acc0)
                    for wv in range(NACC):
                        stage_v[k, po0 + wv] = acc[wv]
                return carry2

            lax.fori_loop(0, NCHUNK, chunk, 0)
            pltpu.sync_copy(stage_v, y_hbm.at[cb, i])

        def job_body(n, carry):
            j = n * NW + wid

            @pl.when(j < NJOBS)
            def _():
                compute(j)
            return carry

        lax.fori_loop(0, NJ, job_body, 0)

    fn = pl.kernel(
        body,
        out_type=jax.ShapeDtypeStruct((BCJ, HO, K, WP, BC_CH), jnp.float32),
        mesh=mesh,
        compiler_params=pltpu.CompilerParams(use_tc_tiling_on_sc=False,
                                             needs_layout_passes=False),
        scratch_types=[
            pltpu.VMEM((S16P,), jnp.int32),
            pltpu.VMEM((S16P,), jnp.float32),
            pltpu.VMEM((HO, L), jnp.int32),
            pltpu.VMEM((2 * NLAT_W, WROW, BC_CH), jnp.float32),
            pltpu.VMEM((K, WP, BC_CH), jnp.float32),
        ],
    )
    return fn(xpar, off, val, ptr2d)


def _tc_einsum_body(w2_ref, y_ref, bias_ref, out_ref):
    res = lax.dot_general(w2_ref[...], y_ref[0],
                          dimension_numbers=(((1,), (0,)), ((), ())),
                          preferred_element_type=jnp.float32)
    out_ref[0] = res + bias_ref[...]


def _tc_einsum(w2, y3, bias2, B, O, CK, NCOL):
    return pl.pallas_call(
        _tc_einsum_body,
        grid=(B,),
        in_specs=[
            pl.BlockSpec((O, CK), lambda b: (0, 0)),
            pl.BlockSpec((1, CK, NCOL), lambda b: (b, 0, 0)),
            pl.BlockSpec((O, 1), lambda b: (0, 0)),
        ],
        out_specs=pl.BlockSpec((1, O, NCOL), lambda b: (b, 0, 0)),
        out_shape=jax.ShapeDtypeStruct((B, O, NCOL), jnp.float32),
    )(w2, y3, bias2)


def kernel(x, psi_ker_idx, psi_row_idx, psi_col_idx, psi_vals, weight, bias):
    B, C, H, W = x.shape
    BC = B * C
    BCJ = BC // BC_CH
    HO = (H + 1) // 2
    O = weight.shape[0]
    NNZ = psi_vals.shape[0]
    NSEG = K * HO
    S16P = -(-(NNZ + NSEG * (L - 1)) // L) * L

    xb = x.reshape(BC, H, WO, 2).transpose(0, 1, 3, 2)
    xpad = jnp.concatenate([xb, xb[..., :NACC]], axis=-1)
    xpar = (xpad.reshape(BCJ, BC_CH, H * 2, WROW)
                .transpose(0, 2, 3, 1))

    lat = psi_col_idx // W
    lon = psi_col_idx % W
    r = lon % 2
    m = lon // 2
    lat0 = jnp.clip(2 * psi_row_idx - 2, 0, H - NLAT_W)
    a = lat - lat0
    off = (((a * 2 + r) << 9) + m).astype(jnp.int32)
    key = (psi_row_idx * K + psi_ker_idx).astype(jnp.int32)
    ptr = jnp.searchsorted(key, jnp.arange(NSEG + 1, dtype=jnp.int32),
                           side="left").astype(jnp.int32)
    nseg = ptr[1:] - ptr[:-1]
    seg16 = -(-nseg // L) * L
    starts16 = jnp.concatenate(
        [jnp.zeros((1,), jnp.int32), jnp.cumsum(seg16).astype(jnp.int32)])
    dst = starts16[key] + (jnp.arange(NNZ, dtype=jnp.int32) - ptr[key])
    offp = jnp.zeros((S16P,), jnp.int32).at[dst].set(off)
    valp = jnp.zeros((S16P,), jnp.float32).at[dst].set(psi_vals)
    ptr2d = jnp.zeros((HO, L), jnp.int32)
    rows4 = (jnp.arange(HO, dtype=jnp.int32)[:, None] * K
             + jnp.arange(K + 1, dtype=jnp.int32)[None, :])
    ptr2d = ptr2d.at[:, : K + 1].set(starts16[rows4])

    y = _sc_sparse_stage(xpar, offp, valp, ptr2d, BC, H, HO)

    w2 = weight.reshape(O, -1)
    CK = w2.shape[1]
    y3 = (y.transpose(0, 4, 2, 1, 3)
           .reshape(B, C * K, HO * WP))
    out = _tc_einsum(w2, y3, bias.reshape(O, 1), B, O, CK, HO * WP)
    return out.reshape(B, O, HO, WP)[..., :WO]


# speedup vs baseline: 1.4970x; 1.4970x over previous
import jax
import jax.numpy as jnp
from jax import lax
from jax.experimental import pallas as pl
from jax.experimental.pallas import tpu as pltpu
from jax.experimental.pallas import tpu_sc as plsc

NC, NS, L = 2, 16, 16
NW = NC * NS
K = 3
WO = 180
WP = 192
NACC = 12
NCHUNK = WO // NACC
WROW = WO + NACC
TMAX = 1536
UNROLL = 8
NLAT_W = 5
BC_CH = 16


def _sc_sparse_stage(xpar, off, val, ptr2d, BC, H, HO):
    S16P = off.shape[0]
    BCJ = BC // BC_CH
    NJOBS = HO * BCJ
    NJ = -(-NJOBS // NW)
    NJ += NJ % 2

    mesh = plsc.VectorSubcoreMesh(core_axis_name="c", subcore_axis_name="s",
                                  num_cores=NC, num_subcores=NS)

    def body(xpar_hbm, off_hbm, val_hbm, ptr_hbm, y_hbm,
             off_v, val_v, ptr_v, xw, stage_v, vsp_v, offs_s):
        wid = lax.axis_index("s") * NC + lax.axis_index("c")
        pltpu.sync_copy(off_hbm, off_v)
        pltpu.sync_copy(val_hbm, val_v)
        pltpu.sync_copy(ptr_hbm, ptr_v)

        def compute(j):
            i = j // BCJ
            cb = j % BCJ
            lat0 = jnp.clip(2 * i - 2, 0, H - NLAT_W)
            pltpu.sync_copy(xpar_hbm.at[cb, pl.ds(2 * lat0, 2 * NLAT_W)], xw)
            pr = ptr_v[i]
            tbase = pr[0]
            nbA = (pr[K] - tbase) >> 4

            def blkA(b, carryA):
                base = tbase + b * L
                offv = off_v[pl.ds(base, L)]
                valv = val_v[pl.ds(base, L)]
                for li in range(L):
                    t = b * L + li
                    offs_s[t] = offv[li]
                    vsp_v[t, :] = jnp.full((L,), valv[li], jnp.float32)
                return carryA

            lax.fori_loop(0, nbA, blkA, 0)

            def chunk(pc, carry2):
                po0 = pc * NACC
                for k in range(K):
                    t0 = pr[k] - tbase
                    nb = (pr[k + 1] - pr[k]) >> 3

                    def blk(b, acc):
                        for li in range(UNROLL):
                            t = t0 + b * UNROLL + li
                            o = offs_s[t]
                            v = vsp_v[t, :]
                            ab = o >> 9
                            mp = (o & 511) + po0
                            mp = jnp.where(mp >= WO, mp - WO, mp)
                            acc = tuple(
                                acc[wv] + v * xw[ab, mp + wv, :]
                                for wv in range(NACC))
                        return acc

                    acc0 = tuple(jnp.zeros((L,), jnp.float32)
                                 for _ in range(NACC))
                    acc = lax.fori_loop(0, nb, blk, acc0)
                    for wv in range(NACC):
                        stage_v[k, po0 + wv] = acc[wv]
                return carry2

            lax.fori_loop(0, NCHUNK, chunk, 0)
            pltpu.sync_copy(stage_v, y_hbm.at[cb, i])

        def job_body(n, carry):
            j = n * NW + wid

            @pl.when(j < NJOBS)
            def _():
                compute(j)
            return carry

        lax.fori_loop(0, NJ, job_body, 0)

    fn = pl.kernel(
        body,
        out_type=jax.ShapeDtypeStruct((BCJ, HO, K, WP, BC_CH), jnp.float32),
        mesh=mesh,
        compiler_params=pltpu.CompilerParams(use_tc_tiling_on_sc=False,
                                             needs_layout_passes=False),
        scratch_types=[
            pltpu.VMEM((S16P,), jnp.int32),
            pltpu.VMEM((S16P,), jnp.float32),
            pltpu.VMEM((HO, L), jnp.int32),
            pltpu.VMEM((2 * NLAT_W, WROW, BC_CH), jnp.float32),
            pltpu.VMEM((K, WP, BC_CH), jnp.float32),
            pltpu.VMEM((TMAX, L), jnp.float32),
            pltpu.SMEM((TMAX,), jnp.int32),
        ],
    )
    return fn(xpar, off, val, ptr2d)


def _tc_einsum_body(w2_ref, y_ref, bias_ref, out_ref):
    res = lax.dot_general(w2_ref[...], y_ref[0],
                          dimension_numbers=(((1,), (0,)), ((), ())),
                          preferred_element_type=jnp.float32)
    out_ref[0] = res + bias_ref[...]


def _tc_einsum(w2, y3, bias2, B, O, CK, NCOL):
    return pl.pallas_call(
        _tc_einsum_body,
        grid=(B,),
        in_specs=[
            pl.BlockSpec((O, CK), lambda b: (0, 0)),
            pl.BlockSpec((1, CK, NCOL), lambda b: (b, 0, 0)),
            pl.BlockSpec((O, 1), lambda b: (0, 0)),
        ],
        out_specs=pl.BlockSpec((1, O, NCOL), lambda b: (b, 0, 0)),
        out_shape=jax.ShapeDtypeStruct((B, O, NCOL), jnp.float32),
    )(w2, y3, bias2)


def kernel(x, psi_ker_idx, psi_row_idx, psi_col_idx, psi_vals, weight, bias):
    B, C, H, W = x.shape
    BC = B * C
    BCJ = BC // BC_CH
    HO = (H + 1) // 2
    O = weight.shape[0]
    NNZ = psi_vals.shape[0]
    NSEG = K * HO
    S16P = -(-(NNZ + NSEG * (L - 1)) // L) * L

    xb = x.reshape(BC, H, WO, 2).transpose(0, 1, 3, 2)
    xpad = jnp.concatenate([xb, xb[..., :NACC]], axis=-1)
    xpar = (xpad.reshape(BCJ, BC_CH, H * 2, WROW)
                .transpose(0, 2, 3, 1))

    lat = psi_col_idx // W
    lon = psi_col_idx % W
    r = lon % 2
    m = lon // 2
    lat0 = jnp.clip(2 * psi_row_idx - 2, 0, H - NLAT_W)
    a = lat - lat0
    off = (((a * 2 + r) << 9) + m).astype(jnp.int32)
    key = (psi_row_idx * K + psi_ker_idx).astype(jnp.int32)
    ptr = jnp.searchsorted(key, jnp.arange(NSEG + 1, dtype=jnp.int32),
                           side="left").astype(jnp.int32)
    nseg = ptr[1:] - ptr[:-1]
    seg16 = -(-nseg // L) * L
    starts16 = jnp.concatenate(
        [jnp.zeros((1,), jnp.int32), jnp.cumsum(seg16).astype(jnp.int32)])
    dst = starts16[key] + (jnp.arange(NNZ, dtype=jnp.int32) - ptr[key])
    offp = jnp.zeros((S16P,), jnp.int32).at[dst].set(off)
    valp = jnp.zeros((S16P,), jnp.float32).at[dst].set(psi_vals)
    ptr2d = jnp.zeros((HO, L), jnp.int32)
    rows4 = (jnp.arange(HO, dtype=jnp.int32)[:, None] * K
             + jnp.arange(K + 1, dtype=jnp.int32)[None, :])
    ptr2d = ptr2d.at[:, : K + 1].set(starts16[rows4])

    y = _sc_sparse_stage(xpar, offp, valp, ptr2d, BC, H, HO)

    w2 = weight.reshape(O, -1)
    CK = w2.shape[1]
    y3 = (y.transpose(0, 4, 2, 1, 3)
           .reshape(B, C * K, HO * WP))
    out = _tc_einsum(w2, y3, bias.reshape(O, 1), B, O, CK, HO * WP)
    return out.reshape(B, O, HO, WP)[..., :WO]


# speedup vs baseline: 2.0857x; 1.3932x over previous
import jax
import jax.numpy as jnp
from jax import lax
from jax.experimental import pallas as pl
from jax.experimental.pallas import tpu as pltpu
from jax.experimental.pallas import tpu_sc as plsc

NC, NS, L = 2, 16, 16
NW = NC * NS
K = 3
WO = 180
WP = 192
NACC = 12
NCHUNK = WO // NACC
WROW = WO + NACC
TMAX = 1536
UNROLL = 8
NLAT_W = 5
BC_CH = 16


HOP = 96


def _sc_sparse_stage(xpar, off, val, ptr2d, B, BC, H, HO, CK):
    S16P = off.shape[0]
    BCJ = BC // BC_CH
    NJOBS = HO * BCJ
    NJ = -(-NJOBS // NW)

    mesh = plsc.VectorSubcoreMesh(core_axis_name="c", subcore_axis_name="s",
                                  num_cores=NC, num_subcores=NS)

    def body(xpar_hbm, off_hbm, val_hbm, ptr_hbm, y_hbm,
             off_v, val_v, ptr_v, xw, stage_v, vsp_v, offs_s):
        wid = lax.axis_index("s") * NC + lax.axis_index("c")
        pltpu.sync_copy(off_hbm, off_v)
        pltpu.sync_copy(val_hbm, val_v)
        pltpu.sync_copy(ptr_hbm, ptr_v)
        qk3 = lax.iota(jnp.int32, L) * K

        def compute(j):
            i = j // BCJ
            cb = j % BCJ
            lat0 = jnp.clip(2 * i - 2, 0, H - NLAT_W)
            pltpu.sync_copy(
                xpar_hbm.at[cb // 8, pl.ds(2 * lat0, 2 * NLAT_W), :,
                            pl.ds((cb % 8) * BC_CH, BC_CH)],
                xw)
            pr = ptr_v[i]
            tbase = pr[0]
            nbA = (pr[K] - tbase) >> 4

            def blkA(b, carryA):
                base = tbase + b * L
                offv = off_v[pl.ds(base, L)]
                valv = val_v[pl.ds(base, L)]
                for li in range(L):
                    t = b * L + li
                    offs_s[t] = offv[li]
                    vsp_v[t, :] = jnp.full((L,), valv[li], jnp.float32)
                return carryA

            lax.fori_loop(0, nbA, blkA, 0)

            def chunk(pc, carry2):
                po0 = pc * NACC
                for k in range(K):
                    t0 = pr[k] - tbase
                    nb = (pr[k + 1] - pr[k]) >> 3

                    def blk(b, acc):
                        for li in range(UNROLL):
                            t = t0 + b * UNROLL + li
                            o = offs_s[t]
                            v = vsp_v[t, :]
                            ab = o >> 9
                            mp = (o & 511) + po0
                            mp = jnp.where(mp >= WO, mp - WO, mp)
                            acc = tuple(
                                acc[wv] + v * xw[ab, mp + wv, :]
                                for wv in range(NACC))
                        return acc

                    acc0 = tuple(jnp.zeros((L,), jnp.float32)
                                 for _ in range(NACC))
                    acc = lax.fori_loop(0, nb, blk, acc0)
                    rowv = qk3 + k
                    for wv in range(NACC):
                        po = po0 + wv
                        plsc.store_scatter(
                            stage_v,
                            [jnp.full((L,), po >> 7, jnp.int32), rowv,
                             jnp.full((L,), po & 127, jnp.int32)],
                            acc[wv])
                return carry2

            lax.fori_loop(0, NCHUNK, chunk, 0)
            b = cb // (BCJ // B)
            ck0 = (cb % (BCJ // B)) * BC_CH * K
            for ph in range(2):
                pltpu.sync_copy(stage_v.at[ph],
                                y_hbm.at[ph, i, b, pl.ds(ck0, BC_CH * K)])

        def job_body(n, carry):
            j = n * NW + wid

            @pl.when(j < NJOBS)
            def _():
                compute(j)
            return carry

        lax.fori_loop(0, NJ, job_body, 0)

    fn = pl.kernel(
        body,
        out_type=jax.ShapeDtypeStruct((2, HOP, B, CK, 128), jnp.float32),
        mesh=mesh,
        compiler_params=pltpu.CompilerParams(use_tc_tiling_on_sc=False,
                                             needs_layout_passes=False),
        scratch_types=[
            pltpu.VMEM((S16P,), jnp.int32),
            pltpu.VMEM((S16P,), jnp.float32),
            pltpu.VMEM((HO, L), jnp.int32),
            pltpu.VMEM((2 * NLAT_W, WROW, BC_CH), jnp.float32),
            pltpu.VMEM((2, BC_CH * K, 128), jnp.float32),
            pltpu.VMEM((TMAX, L), jnp.float32),
            pltpu.SMEM((TMAX,), jnp.int32),
        ],
    )
    return fn(xpar, off, val, ptr2d)


IT = 8


def _tc_einsum_body(w2_ref, y_ref, bias_ref, out_ref):
    w2 = w2_ref[...]
    for tt in range(IT):
        res = lax.dot_general(w2, y_ref[0, tt, 0],
                              dimension_numbers=(((1,), (0,)), ((), ())),
                              preferred_element_type=jnp.float32)
        out_ref[0, 0, :, tt, :] = res + bias_ref[...]


def _tc_einsum(w2, y5, bias2, B, O, CK):
    return pl.pallas_call(
        _tc_einsum_body,
        grid=(2, B, HOP // IT),
        in_specs=[
            pl.BlockSpec((O, CK), lambda ph, b, t: (0, 0)),
            pl.BlockSpec((1, IT, 1, CK, 128), lambda ph, b, t: (ph, t, b, 0, 0)),
            pl.BlockSpec((O, 1), lambda ph, b, t: (0, 0)),
        ],
        out_specs=pl.BlockSpec((1, 1, O, IT, 128),
                               lambda ph, b, t: (ph, b, 0, t, 0)),
        out_shape=jax.ShapeDtypeStruct((2, B, O, HOP, 128), jnp.float32),
    )(w2, y5, bias2)


def kernel(x, psi_ker_idx, psi_row_idx, psi_col_idx, psi_vals, weight, bias):
    B, C, H, W = x.shape
    BC = B * C
    BCJ = BC // BC_CH
    HO = (H + 1) // 2
    O = weight.shape[0]
    NNZ = psi_vals.shape[0]
    NSEG = K * HO
    S16P = -(-(NNZ + NSEG * (L - 1)) // L) * L

    xb = x.reshape(BC, H, WO, 2).transpose(0, 1, 3, 2)
    xpad = jnp.concatenate([xb, xb[..., :NACC]], axis=-1)
    xpar = (xpad.reshape(BC // 128, 128, H * 2, WROW)
                .transpose(0, 2, 3, 1))

    lat = psi_col_idx // W
    lon = psi_col_idx % W
    r = lon % 2
    m = lon // 2
    lat0 = jnp.clip(2 * psi_row_idx - 2, 0, H - NLAT_W)
    a = lat - lat0
    off = (((a * 2 + r) << 9) + m).astype(jnp.int32)
    key = (psi_row_idx * K + psi_ker_idx).astype(jnp.int32)
    ptr = jnp.searchsorted(key, jnp.arange(NSEG + 1, dtype=jnp.int32),
                           side="left").astype(jnp.int32)
    nseg = ptr[1:] - ptr[:-1]
    seg16 = -(-nseg // L) * L
    starts16 = jnp.concatenate(
        [jnp.zeros((1,), jnp.int32), jnp.cumsum(seg16).astype(jnp.int32)])
    dst = starts16[key] + (jnp.arange(NNZ, dtype=jnp.int32) - ptr[key])
    offp = jnp.zeros((S16P,), jnp.int32).at[dst].set(off)
    valp = jnp.zeros((S16P,), jnp.float32).at[dst].set(psi_vals)
    ptr2d = jnp.zeros((HO, L), jnp.int32)
    rows4 = (jnp.arange(HO, dtype=jnp.int32)[:, None] * K
             + jnp.arange(K + 1, dtype=jnp.int32)[None, :])
    ptr2d = ptr2d.at[:, : K + 1].set(starts16[rows4])

    CK = C * K
    y = _sc_sparse_stage(xpar, offp, valp, ptr2d, B, BC, H, HO, CK)

    w2 = weight.reshape(O, CK)
    out5 = _tc_einsum(w2, y, bias.reshape(O, 1), B, O, CK)
    return jnp.concatenate(
        [out5[0, :, :, :HO, :], out5[1, :, :, :HO, : WO - 128]], axis=-1)


# speedup vs baseline: 2.2077x; 1.0585x over previous
import jax
import jax.numpy as jnp
from jax import lax
from jax.experimental import pallas as pl
from jax.experimental.pallas import tpu as pltpu
from jax.experimental.pallas import tpu_sc as plsc

NC, NS, L = 2, 16, 16
NW = NC * NS
K = 3
WO = 180
WP = 192
NACC = 12
NCHUNK = WO // NACC
WROW = WO + NACC
TMAX = 1536
UNROLL = 8
NLAT_W = 5
BC_CH = 16


HOP = 96


def _sc_sparse_stage(xpar, off, val, ptr2d, B, BC, H, HO, CK):
    S16P = off.shape[0]
    BCJ = BC // BC_CH
    NJOBS = HO * BCJ
    NJ = -(-NJOBS // NW)

    mesh = plsc.VectorSubcoreMesh(core_axis_name="c", subcore_axis_name="s",
                                  num_cores=NC, num_subcores=NS)

    def body(xpar_hbm, off_hbm, val_hbm, ptr_hbm, y_hbm,
             off_v, val_v, ptr_v, xw, stage_v, vsp_v, offs_s, xsem):
        wid = lax.axis_index("s") * NC + lax.axis_index("c")
        pltpu.sync_copy(off_hbm, off_v)
        pltpu.sync_copy(val_hbm, val_v)
        pltpu.sync_copy(ptr_hbm, ptr_v)
        qk3 = lax.iota(jnp.int32, L) * K

        def window_src(j):
            jc = jnp.minimum(j, NJOBS - 1)
            i = jc // BCJ
            cb = jc % BCJ
            lat0 = jnp.clip(2 * i - 2, 0, H - NLAT_W)
            return xpar_hbm.at[cb // 8, pl.ds(2 * lat0, 2 * NLAT_W), :,
                               pl.ds((cb % 8) * BC_CH, BC_CH)]

        def compute(j, cur):
            i = j // BCJ
            cb = j % BCJ
            pr = ptr_v[i]
            tbase = pr[0]
            nbA = (pr[K] - tbase) >> 4

            def blkA(b, carryA):
                base = tbase + b * L
                offv = off_v[pl.ds(base, L)]
                valv = val_v[pl.ds(base, L)]
                for li in range(L):
                    t = b * L + li
                    offs_s[t] = offv[li]
                    vsp_v[t, :] = jnp.full((L,), valv[li], jnp.float32)
                return carryA

            lax.fori_loop(0, nbA, blkA, 0)

            def chunk(pc, carry2):
                po0 = pc * NACC
                for k in range(K):
                    t0 = pr[k] - tbase
                    nb = (pr[k + 1] - pr[k]) >> 3

                    def blk(b, acc):
                        for li in range(UNROLL):
                            t = t0 + b * UNROLL + li
                            o = offs_s[t]
                            v = vsp_v[t, :]
                            ab = o >> 9
                            mp = (o & 511) + po0
                            mp = jnp.where(mp >= WO, mp - WO, mp)
                            acc = tuple(
                                acc[wv] + v * xw[cur, ab, mp + wv, :]
                                for wv in range(NACC))
                        return acc

                    acc0 = tuple(jnp.zeros((L,), jnp.float32)
                                 for _ in range(NACC))
                    acc = lax.fori_loop(0, nb, blk, acc0)
                    rowv = qk3 + k
                    for wv in range(NACC):
                        po = po0 + wv
                        plsc.store_scatter(
                            stage_v,
                            [jnp.full((L,), po >> 7, jnp.int32), rowv,
                             jnp.full((L,), po & 127, jnp.int32)],
                            acc[wv])
                return carry2

            lax.fori_loop(0, NCHUNK, chunk, 0)
            b = cb // (BCJ // B)
            ck0 = (cb % (BCJ // B)) * BC_CH * K
            for ph in range(2):
                pltpu.sync_copy(stage_v.at[ph],
                                y_hbm.at[ph, i, b, pl.ds(ck0, BC_CH * K)])

        pltpu.async_copy(window_src(wid), xw.at[0], xsem)

        def job_body(n, carry):
            j = n * NW + wid
            cur = n & 1
            pltpu.make_async_copy(window_src(j), xw.at[cur], xsem).wait()
            pltpu.async_copy(window_src(j + NW), xw.at[1 - cur], xsem)

            @pl.when(j < NJOBS)
            def _():
                compute(j, cur)
            return carry

        lax.fori_loop(0, NJ, job_body, 0)
        pltpu.make_async_copy(window_src(0), xw.at[NJ & 1], xsem).wait()

    fn = pl.kernel(
        body,
        out_type=jax.ShapeDtypeStruct((2, HOP, B, CK, 128), jnp.float32),
        mesh=mesh,
        compiler_params=pltpu.CompilerParams(use_tc_tiling_on_sc=False,
                                             needs_layout_passes=False),
        scratch_types=[
            pltpu.VMEM((S16P,), jnp.int32),
            pltpu.VMEM((S16P,), jnp.float32),
            pltpu.VMEM((HO, L), jnp.int32),
            pltpu.VMEM((2, 2 * NLAT_W, WROW, BC_CH), jnp.float32),
            pltpu.VMEM((2, BC_CH * K, 128), jnp.float32),
            pltpu.VMEM((TMAX, L), jnp.float32),
            pltpu.SMEM((TMAX,), jnp.int32),
            pltpu.SemaphoreType.DMA,
        ],
    )
    return fn(xpar, off, val, ptr2d)


IT = 8


def _tc_einsum_body(w2_ref, y_ref, bias_ref, out_ref):
    w2 = w2_ref[...]
    for tt in range(IT):
        res = lax.dot_general(w2, y_ref[0, tt, 0],
                              dimension_numbers=(((1,), (0,)), ((), ())),
                              preferred_element_type=jnp.float32)
        out_ref[0, 0, :, tt, :] = res + bias_ref[...]


def _tc_einsum(w2, y5, bias2, B, O, CK):
    return pl.pallas_call(
        _tc_einsum_body,
        grid=(2, B, HOP // IT),
        in_specs=[
            pl.BlockSpec((O, CK), lambda ph, b, t: (0, 0)),
            pl.BlockSpec((1, IT, 1, CK, 128), lambda ph, b, t: (ph, t, b, 0, 0)),
            pl.BlockSpec((O, 1), lambda ph, b, t: (0, 0)),
        ],
        out_specs=pl.BlockSpec((1, 1, O, IT, 128),
                               lambda ph, b, t: (ph, b, 0, t, 0)),
        out_shape=jax.ShapeDtypeStruct((2, B, O, HOP, 128), jnp.float32),
    )(w2, y5, bias2)


def kernel(x, psi_ker_idx, psi_row_idx, psi_col_idx, psi_vals, weight, bias):
    B, C, H, W = x.shape
    BC = B * C
    BCJ = BC // BC_CH
    HO = (H + 1) // 2
    O = weight.shape[0]
    NNZ = psi_vals.shape[0]
    NSEG = K * HO
    S16P = -(-(NNZ + NSEG * (L - 1)) // L) * L

    xb = x.reshape(BC, H, WO, 2).transpose(0, 1, 3, 2)
    xpad = jnp.concatenate([xb, xb[..., :NACC]], axis=-1)
    xpar = (xpad.reshape(BC // 128, 128, H * 2, WROW)
                .transpose(0, 2, 3, 1))

    lat = psi_col_idx // W
    lon = psi_col_idx % W
    r = lon % 2
    m = lon // 2
    lat0 = jnp.clip(2 * psi_row_idx - 2, 0, H - NLAT_W)
    a = lat - lat0
    off = (((a * 2 + r) << 9) + m).astype(jnp.int32)
    key = (psi_row_idx * K + psi_ker_idx).astype(jnp.int32)
    ptr = jnp.searchsorted(key, jnp.arange(NSEG + 1, dtype=jnp.int32),
                           side="left").astype(jnp.int32)
    nseg = ptr[1:] - ptr[:-1]
    seg16 = -(-nseg // L) * L
    starts16 = jnp.concatenate(
        [jnp.zeros((1,), jnp.int32), jnp.cumsum(seg16).astype(jnp.int32)])
    dst = starts16[key] + (jnp.arange(NNZ, dtype=jnp.int32) - ptr[key])
    offp = jnp.zeros((S16P,), jnp.int32).at[dst].set(off)
    valp = jnp.zeros((S16P,), jnp.float32).at[dst].set(psi_vals)
    ptr2d = jnp.zeros((HO, L), jnp.int32)
    rows4 = (jnp.arange(HO, dtype=jnp.int32)[:, None] * K
             + jnp.arange(K + 1, dtype=jnp.int32)[None, :])
    ptr2d = ptr2d.at[:, : K + 1].set(starts16[rows4])

    CK = C * K
    y = _sc_sparse_stage(xpar, offp, valp, ptr2d, B, BC, H, HO, CK)

    w2 = weight.reshape(O, CK)
    out5 = _tc_einsum(w2, y, bias.reshape(O, 1), B, O, CK)
    return jnp.concatenate(
        [out5[0, :, :, :HO, :], out5[1, :, :, :HO, : WO - 128]], axis=-1)


# speedup vs baseline: 2.9980x; 1.3579x over previous
import jax
import jax.numpy as jnp
from jax import lax
from jax.experimental import pallas as pl
from jax.experimental.pallas import tpu as pltpu
from jax.experimental.pallas import tpu_sc as plsc

NC, NS, L = 2, 16, 16
NW = NC * NS
K = 3
WO = 180
WP = 192
NACC = 12
NCHUNK = WO // NACC
WROW = WO + NACC
TMAX = 1536
UNROLL = 8
NLAT_W = 5
BC_CH = 16


HOP = 96


def _sc_sparse_stage(xpar, off, val, ptr2d, B, BC, H, HO, CK):
    S16P = off.shape[0]
    BCJ = BC // BC_CH
    NJOBS = HO * BCJ
    NJ = -(-NJOBS // NW)

    mesh = plsc.VectorSubcoreMesh(core_axis_name="c", subcore_axis_name="s",
                                  num_cores=NC, num_subcores=NS)

    def body(xpar_hbm, off_hbm, val_hbm, ptr_hbm, y_hbm,
             off_v, val_v, ptr_v, xw, stage_v, vsp_v, offs_s, xsem):
        wid = lax.axis_index("s") * NC + lax.axis_index("c")
        pltpu.sync_copy(off_hbm, off_v)
        pltpu.sync_copy(val_hbm, val_v)
        pltpu.sync_copy(ptr_hbm, ptr_v)
        qk3 = lax.iota(jnp.int32, L) * K

        def window_src(j):
            jc = jnp.minimum(j, NJOBS - 1)
            i = jc // BCJ
            cb = jc % BCJ
            lat0 = jnp.clip(2 * i - 2, 0, H - NLAT_W)
            return xpar_hbm.at[cb // 8, pl.ds(2 * lat0, 2 * NLAT_W), :,
                               pl.ds((cb % 8) * BC_CH, BC_CH)]

        def compute(j, cur):
            i = j // BCJ
            cb = j % BCJ
            pr = ptr_v[i]
            tbase = pr[0]
            nbA = (pr[K] - tbase) >> 4

            def blkA(b, carryA):
                base = tbase + b * L
                offv = off_v[pl.ds(base, L)]
                valv = val_v[pl.ds(base, L)]
                for li in range(L):
                    t = b * L + li
                    offs_s[t] = offv[li]
                    vsp_v[t, :] = jnp.full((L,), valv[li], jnp.float32)
                return carryA

            lax.fori_loop(0, nbA, blkA, 0)

            def chunk(pc, carry2):
                po0 = pc * NACC
                for k in range(K):
                    t0 = pr[k] - tbase
                    nb = (pr[k + 1] - pr[k]) >> 3

                    def blk(b, acc):
                        for li in range(UNROLL):
                            t = t0 + b * UNROLL + li
                            o = offs_s[t]
                            v = vsp_v[t, :]
                            ab = o >> 9
                            mp = (o & 511) + po0
                            mp = jnp.where(mp >= WO, mp - WO, mp)
                            acc = tuple(
                                acc[wv] + v * xw[cur, ab, mp + wv, :]
                                for wv in range(NACC))
                        return acc

                    acc0 = tuple(jnp.zeros((L,), jnp.float32)
                                 for _ in range(NACC))
                    acc = lax.fori_loop(0, nb, blk, acc0)
                    rowv = qk3 + k
                    for wv in range(NACC):
                        po = po0 + wv
                        plsc.store_scatter(
                            stage_v,
                            [jnp.full((L,), po >> 7, jnp.int32), rowv,
                             jnp.full((L,), po & 127, jnp.int32)],
                            acc[wv])
                return carry2

            lax.fori_loop(0, NCHUNK, chunk, 0)
            b = cb // (BCJ // B)
            ck0 = (cb % (BCJ // B)) * BC_CH * K
            for ph in range(2):
                pltpu.sync_copy(stage_v.at[ph],
                                y_hbm.at[ph, i, b, pl.ds(ck0, BC_CH * K)])

        pltpu.async_copy(window_src(wid), xw.at[0], xsem)

        def job_body(n, carry):
            j = n * NW + wid
            cur = n & 1
            pltpu.make_async_copy(window_src(j), xw.at[cur], xsem).wait()
            pltpu.async_copy(window_src(j + NW), xw.at[1 - cur], xsem)

            @pl.when(j < NJOBS)
            def _():
                compute(j, cur)
            return carry

        lax.fori_loop(0, NJ, job_body, 0)
        pltpu.make_async_copy(window_src(0), xw.at[NJ & 1], xsem).wait()

    fn = pl.kernel(
        body,
        out_type=jax.ShapeDtypeStruct((2, HOP, B, CK, 128), jnp.float32),
        mesh=mesh,
        compiler_params=pltpu.CompilerParams(use_tc_tiling_on_sc=False,
                                             needs_layout_passes=False),
        scratch_types=[
            pltpu.VMEM((S16P,), jnp.int32),
            pltpu.VMEM((S16P,), jnp.float32),
            pltpu.VMEM((HO, L), jnp.int32),
            pltpu.VMEM((2, 2 * NLAT_W, WROW, BC_CH), jnp.float32),
            pltpu.VMEM((2, BC_CH * K, 128), jnp.float32),
            pltpu.VMEM((TMAX, L), jnp.float32),
            pltpu.SMEM((TMAX,), jnp.int32),
            pltpu.SemaphoreType.DMA,
        ],
    )
    return fn(xpar, off, val, ptr2d)


HB = 8


def _tc_xpar_body(x_ref, out_ref):
    xb = x_ref[0]
    for hh in range(HB):
        t = jnp.transpose(xb[:, hh, :])
        t3 = t.reshape(WO, 2, 128)
        for r in range(2):
            xr = t3[:, r, :]
            out_ref[0, 2 * hh + r, :WO, :] = xr
            out_ref[0, 2 * hh + r, WO:, :] = xr[:NACC, :]


def _tc_xpar(x4, G, H, W):
    HT = -(-H // HB)
    return pl.pallas_call(
        _tc_xpar_body,
        grid=(G, HT),
        in_specs=[pl.BlockSpec((1, 128, HB, W), lambda g, t: (g, 0, t, 0))],
        out_specs=pl.BlockSpec((1, 2 * HB, WROW, 128),
                               lambda g, t: (g, t, 0, 0)),
        out_shape=jax.ShapeDtypeStruct((G, HT * HB * 2, WROW, 128),
                                       jnp.float32),
    )(x4)


IT = 8


def _tc_einsum_body(w2_ref, y_ref, bias_ref, out_ref):
    w2 = w2_ref[...]
    for tt in range(IT):
        res = lax.dot_general(w2, y_ref[0, tt, 0],
                              dimension_numbers=(((1,), (0,)), ((), ())),
                              preferred_element_type=jnp.float32)
        out_ref[0, 0, :, tt, :] = res + bias_ref[...]


def _tc_einsum(w2, y5, bias2, B, O, CK):
    return pl.pallas_call(
        _tc_einsum_body,
        grid=(2, B, HOP // IT),
        in_specs=[
            pl.BlockSpec((O, CK), lambda ph, b, t: (0, 0)),
            pl.BlockSpec((1, IT, 1, CK, 128), lambda ph, b, t: (ph, t, b, 0, 0)),
            pl.BlockSpec((O, 1), lambda ph, b, t: (0, 0)),
        ],
        out_specs=pl.BlockSpec((1, 1, O, IT, 128),
                               lambda ph, b, t: (ph, b, 0, t, 0)),
        out_shape=jax.ShapeDtypeStruct((2, B, O, HOP, 128), jnp.float32),
    )(w2, y5, bias2)


def kernel(x, psi_ker_idx, psi_row_idx, psi_col_idx, psi_vals, weight, bias):
    B, C, H, W = x.shape
    BC = B * C
    BCJ = BC // BC_CH
    HO = (H + 1) // 2
    O = weight.shape[0]
    NNZ = psi_vals.shape[0]
    NSEG = K * HO
    S16P = -(-(NNZ + NSEG * (L - 1)) // L) * L

    xpar = _tc_xpar(x.reshape(BC // 128, 128, H, W), BC // 128, H, W)

    lat = psi_col_idx // W
    lon = psi_col_idx % W
    r = lon % 2
    m = lon // 2
    lat0 = jnp.clip(2 * psi_row_idx - 2, 0, H - NLAT_W)
    a = lat - lat0
    off = (((a * 2 + r) << 9) + m).astype(jnp.int32)
    key = (psi_row_idx * K + psi_ker_idx).astype(jnp.int32)
    ptr = jnp.searchsorted(key, jnp.arange(NSEG + 1, dtype=jnp.int32),
                           side="left").astype(jnp.int32)
    nseg = ptr[1:] - ptr[:-1]
    seg16 = -(-nseg // L) * L
    starts16 = jnp.concatenate(
        [jnp.zeros((1,), jnp.int32), jnp.cumsum(seg16).astype(jnp.int32)])
    dst = starts16[key] + (jnp.arange(NNZ, dtype=jnp.int32) - ptr[key])
    offp = jnp.zeros((S16P,), jnp.int32).at[dst].set(off)
    valp = jnp.zeros((S16P,), jnp.float32).at[dst].set(psi_vals)
    ptr2d = jnp.zeros((HO, L), jnp.int32)
    rows4 = (jnp.arange(HO, dtype=jnp.int32)[:, None] * K
             + jnp.arange(K + 1, dtype=jnp.int32)[None, :])
    ptr2d = ptr2d.at[:, : K + 1].set(starts16[rows4])

    CK = C * K
    y = _sc_sparse_stage(xpar, offp, valp, ptr2d, B, BC, H, HO, CK)

    w2 = weight.reshape(O, CK)
    out5 = _tc_einsum(w2, y, bias.reshape(O, 1), B, O, CK)
    return jnp.concatenate(
        [out5[0, :, :, :HO, :], out5[1, :, :, :HO, : WO - 128]], axis=-1)
